# baseline (speedup 1.0000x reference)
import functools

import jax
import jax.numpy as jnp
from jax import lax
from jax.experimental import pallas as pl
from jax.experimental.pallas import tpu as pltpu
from jax.experimental.pallas import tpu_sc as plsc

H_DIM = 64
GRID = 8
G2 = GRID * GRID
NBHD = 2.0
BOTTLENECK = 1024
NUM_SEQS = 32
PEDS = 128
BATCH = NUM_SEQS * PEDS
MBLK = 512
LANES = 16
NS = 16
SLAB = 40
DUMP = 32
NBUF = 4


def _idx_body(xq_ref, yq_ref, xp_ref, yp_ref, oute_ref, outo_ref):
    xp = xp_ref[...]
    yp = yp_ref[...]
    xq = xq_ref[0]
    yq = yq_ref[0]
    tlx = xp - NBHD / 2
    tly = yp + NBHD / 2
    brx = xp + NBHD / 2
    bry = yp - NBHD / 2
    cx = jnp.floor((xq - tlx) / NBHD * GRID)
    cy = jnp.floor((tly - yq) / NBHD * GRID)
    cell = (cx + cy * GRID).astype(jnp.int32)
    oob_x = (xq >= brx) | (xq <= tlx)
    oob_y = (yq >= tly) | (yq <= bry)
    pp = jax.lax.broadcasted_iota(jnp.int32, (PEDS, PEDS), 0)
    qq = jax.lax.broadcasted_iota(jnp.int32, (PEDS, PEDS), 1)
    valid = jnp.logical_not(oob_x | oob_y) & (pp != qq)
    half = cell // 2
    even = valid & ((cell % 2) == 0)
    odd = valid & ((cell % 2) == 1)
    oute_ref[...] = jnp.where(even, half, DUMP)
    outo_ref[...] = jnp.where(odd, half, DUMP)


def _sc_pool_body(he_hbm, ho_hbm, ie_hbm, io_hbm, out_hbm,
                  ch2_v, ime_v, imo_v, zero_v, id2_v, pool_sh):
    cid = lax.axis_index("c")
    sid = lax.axis_index("s")
    wid = cid * NS + sid
    base = wid * PEDS

    pltpu.sync_copy(he_hbm.at[pl.ds(base, PEDS)], ch2_v.at[pl.ds(0, PEDS)])
    pltpu.sync_copy(ho_hbm.at[pl.ds(base, PEDS)],
                    ch2_v.at[pl.ds(PEDS, PEDS)])
    pltpu.sync_copy(ie_hbm.at[pl.ds(base, PEDS)], ime_v)
    pltpu.sync_copy(io_hbm.at[pl.ds(base, PEDS)], imo_v)

    z16 = jnp.zeros((LANES,), jnp.float32)

    def zrow(r, _):
        for k in range(2 * H_DIM // LANES):
            zero_v[r, pl.ds(k * LANES, LANES)] = z16
        return 0

    lax.fori_loop(0, SLAB, zrow, 0)
    for r in range(NBUF):
        pltpu.sync_copy(zero_v,
                        pool_sh.at[pl.ds((sid * NBUF + r) * SLAB, SLAB)])
    off0 = jnp.full((LANES,), sid * NBUF * SLAB, jnp.int32)

    def drain(p):
        sbase = sid * NBUF * SLAB + (p % NBUF) * SLAB
        pltpu.sync_copy(pool_sh.at[pl.ds(sbase, DUMP)],
                        out_hbm.at[pl.ds((base + p) * DUMP, DUMP)])
        pltpu.sync_copy(zero_v, pool_sh.at[pl.ds(sbase, SLAB)])

    def anchor(p, _):
        off = off0 + (p % NBUF) * SLAB
        for k in range(PEDS // LANES):
            id2_v[pl.ds(k * LANES, LANES)] = (
                ime_v[p, pl.ds(k * LANES, LANES)] + off)
            id2_v[pl.ds(PEDS + k * LANES, LANES)] = (
                imo_v[p, pl.ds(k * LANES, LANES)] + off)
        pltpu.sync_copy(ch2_v, pool_sh.at[id2_v], add=True)

        @pl.when(p != 0)
        def _():
            drain(p - 1)

        return 0

    lax.fori_loop(0, PEDS, anchor, 0)
    drain(PEDS - 1)


def _sc_pool(h_even, h_odd, idx_even, idx_odd):
    mesh = plsc.VectorSubcoreMesh(core_axis_name="c", subcore_axis_name="s")
    fn = functools.partial(
        pl.kernel,
        out_type=jax.ShapeDtypeStruct((BATCH * DUMP, 2 * H_DIM), jnp.float32),
        mesh=mesh,
        scratch_types=[
            pltpu.VMEM((2 * PEDS, 2 * H_DIM), jnp.float32),
            pltpu.VMEM((PEDS, PEDS), jnp.int32),
            pltpu.VMEM((PEDS, PEDS), jnp.int32),
            pltpu.VMEM((SLAB, 2 * H_DIM), jnp.float32),
            pltpu.VMEM((2 * PEDS,), jnp.int32),
            pltpu.VMEM_SHARED((NS * NBUF * SLAB, 2 * H_DIM), jnp.float32),
        ],
    )(_sc_pool_body)
    return fn(h_even, h_odd, idx_even, idx_odd)


def _proj_body(x_ref, w_ref, y_ref, s_ref, s2_ref, wb_ref):
    i = pl.program_id(0)

    @pl.when(i == 0)
    def _():
        wb_ref[...] = w_ref[...].astype(jnp.bfloat16)

    y = jax.lax.dot_general(
        x_ref[...].astype(jnp.bfloat16), wb_ref[...], (((1,), (1,)), ((), ())),
        preferred_element_type=jnp.float32)
    y_ref[...] = y.astype(jnp.bfloat16)
    ps = jnp.sum(y, axis=0, keepdims=True)
    ps2 = jnp.sum(y * y, axis=0, keepdims=True)

    @pl.when(i == 0)
    def _():
        s_ref[...] = ps
        s2_ref[...] = ps2

    @pl.when(i != 0)
    def _():
        s_ref[...] += ps
        s2_ref[...] += ps2


def _bn_body(y_ref, s_ref, s2_ref, g_ref, bt_ref, out_ref):
    mean = s_ref[...] * (1.0 / BATCH)
    ex2 = s2_ref[...] * (1.0 / BATCH)
    var = ex2 - mean * mean
    inv = jax.lax.rsqrt(var + 1e-5)
    yn = (y_ref[...].astype(jnp.float32) - mean) * inv * g_ref[...] \
        + bt_ref[...]
    out_ref[...] = jnp.maximum(yn, 0.0)


@functools.partial(jax.jit, static_argnames=())
def kernel(h_states, seq_start_end, end_pos, rel_pos, W, b, gamma, beta):
    del seq_start_end, rel_pos
    h_flat = h_states.reshape(BATCH, H_DIM)
    zpad = jnp.zeros((BATCH, H_DIM), jnp.float32)
    h_even = jnp.concatenate([h_flat, zpad], axis=1)
    h_odd = jnp.concatenate([zpad, h_flat], axis=1)
    xq = end_pos[:, 0].reshape(NUM_SEQS, 1, PEDS)
    yq = end_pos[:, 1].reshape(NUM_SEQS, 1, PEDS)
    xp = end_pos[:, 0].reshape(BATCH, 1)
    yp = end_pos[:, 1].reshape(BATCH, 1)

    idx_even, idx_odd = pl.pallas_call(
        _idx_body,
        grid=(NUM_SEQS,),
        in_specs=[
            pl.BlockSpec((1, 1, PEDS), lambda i: (i, 0, 0)),
            pl.BlockSpec((1, 1, PEDS), lambda i: (i, 0, 0)),
            pl.BlockSpec((PEDS, 1), lambda i: (i, 0)),
            pl.BlockSpec((PEDS, 1), lambda i: (i, 0)),
        ],
        out_specs=[
            pl.BlockSpec((PEDS, PEDS), lambda i: (i, 0)),
            pl.BlockSpec((PEDS, PEDS), lambda i: (i, 0)),
        ],
        out_shape=[
            jax.ShapeDtypeStruct((BATCH, PEDS), jnp.int32),
            jax.ShapeDtypeStruct((BATCH, PEDS), jnp.int32),
        ],
    )(xq, yq, xp, yp)

    pool = _sc_pool(h_even, h_odd, idx_even, idx_odd)
    pool_h = pool.reshape(BATCH, G2 * H_DIM)

    y_raw, s, s2 = pl.pallas_call(
        _proj_body,
        grid=(BATCH // MBLK,),
        in_specs=[
            pl.BlockSpec((MBLK, G2 * H_DIM), lambda i: (i, 0)),
            pl.BlockSpec((BOTTLENECK, G2 * H_DIM), lambda i: (0, 0)),
        ],
        out_specs=[
            pl.BlockSpec((MBLK, BOTTLENECK), lambda i: (i, 0)),
            pl.BlockSpec((1, BOTTLENECK), lambda i: (0, 0)),
            pl.BlockSpec((1, BOTTLENECK), lambda i: (0, 0)),
        ],
        out_shape=[
            jax.ShapeDtypeStruct((BATCH, BOTTLENECK), jnp.bfloat16),
            jax.ShapeDtypeStruct((1, BOTTLENECK), jnp.float32),
            jax.ShapeDtypeStruct((1, BOTTLENECK), jnp.float32),
        ],
        scratch_shapes=[pltpu.VMEM((BOTTLENECK, G2 * H_DIM), jnp.bfloat16)],
    )(pool_h, W)

    out = pl.pallas_call(
        _bn_body,
        grid=(BATCH // MBLK,),
        in_specs=[
            pl.BlockSpec((MBLK, BOTTLENECK), lambda i: (i, 0)),
            pl.BlockSpec((1, BOTTLENECK), lambda i: (0, 0)),
            pl.BlockSpec((1, BOTTLENECK), lambda i: (0, 0)),
            pl.BlockSpec((1, BOTTLENECK), lambda i: (0, 0)),
            pl.BlockSpec((1, BOTTLENECK), lambda i: (0, 0)),
        ],
        out_specs=pl.BlockSpec((MBLK, BOTTLENECK), lambda i: (i, 0)),
        out_shape=jax.ShapeDtypeStruct((BATCH, BOTTLENECK), jnp.float32),
    )(y_raw, s, s2, gamma.reshape(1, BOTTLENECK), beta.reshape(1, BOTTLENECK))
    return out
